# baseline (speedup 1.0000x reference)
import jax
import jax.numpy as jnp
from jax.experimental import pallas as pl


def _copy_kernel(experts_ref, scales_ref, experts_out_ref, scales_out_ref):
    experts_out_ref[...] = experts_ref[...]
    scales_out_ref[...] = scales_ref[...]


def kernel(router_logits, routing_tensor, routing_scales):
    del router_logits
    n_tokens, top_k = routing_tensor.shape
    rows = (n_tokens * top_k) // 128
    experts2d = routing_tensor.reshape(rows, 128)
    scales2d = routing_scales.reshape(rows, 128)
    experts_out, scales_out = pl.pallas_call(
        _copy_kernel,
        out_shape=(
            jax.ShapeDtypeStruct((rows, 128), routing_tensor.dtype),
            jax.ShapeDtypeStruct((rows, 128), routing_scales.dtype),
        ),
    )(experts2d, scales2d)
    return (
        experts_out.reshape(n_tokens, top_k),
        scales_out.reshape(n_tokens, top_k),
    )
